# baseline (speedup 1.0000x reference)
import functools

import jax
import jax.numpy as jnp
from jax import lax
from jax.experimental import pallas as pl
from jax.experimental.pallas import tpu as pltpu
from jax.experimental.pallas import tpu_sc as plsc

N_NODES = 10000
N_EDGES = 160000
D_FEAT = 256
D_EDGE = 16
D_HID = 512
D_OUT = 256

NC = 2
NS = 16
NW = NC * NS
B_PER_W = N_EDGES // NW
CHUNK = 40
NCHUNK = B_PER_W // CHUNK

FMIN = float(jnp.finfo(jnp.float32).min)


def _sc_gather(table, idx3):
    D = table.shape[1]
    mesh = plsc.VectorSubcoreMesh(core_axis_name="c", subcore_axis_name="s")

    @functools.partial(
        pl.kernel,
        mesh=mesh,
        out_type=jax.ShapeDtypeStruct((N_EDGES, D), jnp.float32),
        scratch_types=[
            pltpu.VMEM((NCHUNK, CHUNK), jnp.int32),
            pltpu.VMEM((CHUNK, D), jnp.float32),
            pltpu.SemaphoreType.DMA,
        ],
    )
    def gather_k(table_hbm, idx_hbm, out_hbm, idx_v, buf, sem):
        wid = lax.axis_index("s") * NC + lax.axis_index("c")
        base = wid * B_PER_W
        pltpu.sync_copy(idx_hbm.at[wid], idx_v)

        def body(c, carry):
            pltpu.async_copy(table_hbm.at[idx_v.at[c]], buf, sem).wait()
            pltpu.sync_copy(buf, out_hbm.at[pl.ds(base + c * CHUNK, CHUNK)])
            return carry

        lax.fori_loop(0, NCHUNK, body, 0)

    return gather_k(table, idx3)


def _mlp_body(g_ref, e_ref, w1a_ref, w1b_ref, b1_ref, w2_ref, b2_ref, o_ref):
    h = jnp.dot(g_ref[...], w1a_ref[...], preferred_element_type=jnp.float32)
    h = h + jnp.dot(e_ref[...], w1b_ref[...], preferred_element_type=jnp.float32)
    h = jnp.maximum(h + b1_ref[...], 0.0)
    o_ref[...] = jnp.dot(h, w2_ref[...], preferred_element_type=jnp.float32) + b2_ref[...]


def _edge_mlp(gathered, edges, W1a, W1b, b1, W2, b2):
    BLK = 640
    grid = (N_EDGES // BLK,)
    return pl.pallas_call(
        _mlp_body,
        grid=grid,
        in_specs=[
            pl.BlockSpec((BLK, D_FEAT), lambda i: (i, 0)),
            pl.BlockSpec((BLK, D_EDGE), lambda i: (i, 0)),
            pl.BlockSpec((D_FEAT, D_HID), lambda i: (0, 0)),
            pl.BlockSpec((D_EDGE, D_HID), lambda i: (0, 0)),
            pl.BlockSpec((1, D_HID), lambda i: (0, 0)),
            pl.BlockSpec((D_HID, D_OUT), lambda i: (0, 0)),
            pl.BlockSpec((1, D_OUT), lambda i: (0, 0)),
        ],
        out_specs=pl.BlockSpec((BLK, D_OUT), lambda i: (i, 0)),
        out_shape=jax.ShapeDtypeStruct((N_EDGES, D_OUT), jnp.float32),
    )(gathered, edges, W1a, W1b, b1, W2, b2)


def _smax_body(rec_ref, rows_ref, o_ref):
    EB = rows_ref.shape[0]

    @pl.when(pl.program_id(0) == 0)
    def _():
        o_ref[...] = jnp.full_like(o_ref[...], FMIN)

    def body(j, carry):
        r = rec_ref[0, 0, j]
        row = rows_ref[pl.ds(j, 1), :]
        o_ref[pl.ds(r, 1), :] = jnp.maximum(o_ref[pl.ds(r, 1), :], row)
        return carry

    lax.fori_loop(0, EB, body, 0)


def _segment_max(new_edges, receivers):
    EB = 256
    grid = (N_EDGES // EB,)
    rec3 = receivers.reshape(N_EDGES // EB, 1, EB)
    return pl.pallas_call(
        _smax_body,
        grid=grid,
        in_specs=[
            pl.BlockSpec((1, 1, EB), lambda i: (i, 0, 0), memory_space=pltpu.SMEM),
            pl.BlockSpec((EB, D_OUT), lambda i: (i, 0)),
        ],
        out_specs=pl.BlockSpec((N_NODES, D_OUT), lambda i: (0, 0)),
        out_shape=jax.ShapeDtypeStruct((N_NODES, D_OUT), jnp.float32),
    )(rec3, new_edges)


def kernel(nodes, edges, senders, receivers, W1, b1, W2, b2):
    idx3 = senders.reshape(NW, NCHUNK, CHUNK)
    gathered = _sc_gather(nodes, idx3)
    new_edges = _edge_mlp(
        gathered, edges,
        W1[:D_FEAT], W1[D_FEAT:],
        b1.reshape(1, D_HID), W2, b2.reshape(1, D_OUT),
    )
    return _segment_max(new_edges, receivers)


# speedup vs baseline: 1.0433x; 1.0433x over previous
import functools

import jax
import jax.numpy as jnp
from jax import lax
from jax.experimental import pallas as pl
from jax.experimental.pallas import tpu as pltpu
from jax.experimental.pallas import tpu_sc as plsc

N_NODES = 10000
N_EDGES = 160000
D_FEAT = 256
D_EDGE = 16
D_HID = 512
D_OUT = 256

NC = 2
NS = 16
NW = NC * NS
B_PER_W = N_EDGES // NW
CHUNK = 40
NCHUNK = B_PER_W // CHUNK

FMIN = float(jnp.finfo(jnp.float32).min)


def _sc_gather(table, idx3):
    D = table.shape[1]
    mesh = plsc.VectorSubcoreMesh(core_axis_name="c", subcore_axis_name="s")

    NBUF = 4
    NGRP = (NCHUNK + NBUF - 1) // NBUF

    @functools.partial(
        pl.kernel,
        mesh=mesh,
        out_type=jax.ShapeDtypeStruct((N_EDGES, D), jnp.float32),
        scratch_types=[
            pltpu.VMEM((NCHUNK, CHUNK), jnp.int32),
            pltpu.VMEM((CHUNK, D), jnp.float32),
            pltpu.VMEM((CHUNK, D), jnp.float32),
            pltpu.VMEM((CHUNK, D), jnp.float32),
            pltpu.VMEM((CHUNK, D), jnp.float32),
            pltpu.SemaphoreType.DMA,
            pltpu.SemaphoreType.DMA,
            pltpu.SemaphoreType.DMA,
            pltpu.SemaphoreType.DMA,
        ],
    )
    def gather_k(table_hbm, idx_hbm, out_hbm, idx_v,
                 buf0, buf1, buf2, buf3, sem0, sem1, sem2, sem3):
        bufs = (buf0, buf1, buf2, buf3)
        sems = (sem0, sem1, sem2, sem3)
        wid = lax.axis_index("s") * NC + lax.axis_index("c")
        base = wid * B_PER_W
        pltpu.sync_copy(idx_hbm.at[wid], idx_v)

        for b in range(NBUF):
            pltpu.async_copy(table_hbm.at[idx_v.at[b]], bufs[b], sems[b])

        def body(g, carry):
            for b in range(NBUF):
                c = g * NBUF + b

                @pl.when(c < NCHUNK)
                def _():
                    pltpu.make_async_copy(
                        out_hbm.at[pl.ds(0, CHUNK)], bufs[b], sems[b]).wait()
                    pltpu.sync_copy(
                        bufs[b], out_hbm.at[pl.ds(base + c * CHUNK, CHUNK)])

                @pl.when(c + NBUF < NCHUNK)
                def _():
                    pltpu.async_copy(
                        table_hbm.at[idx_v.at[c + NBUF]], bufs[b], sems[b])
            return carry

        lax.fori_loop(0, NGRP, body, 0)

    return gather_k(table, idx3)


def _mlp_body(g_ref, e_ref, w1a_ref, w1b_ref, b1_ref, w2_ref, b2_ref, o_ref):
    g16 = g_ref[...].astype(jnp.bfloat16)
    w1a16 = w1a_ref[...].astype(jnp.bfloat16)
    h = jnp.dot(g16, w1a16, preferred_element_type=jnp.float32)
    h = h + jnp.dot(e_ref[...], w1b_ref[...], preferred_element_type=jnp.float32)
    h = jnp.maximum(h + b1_ref[...], 0.0)
    o_ref[...] = jnp.dot(h.astype(jnp.bfloat16), w2_ref[...].astype(jnp.bfloat16),
                         preferred_element_type=jnp.float32) + b2_ref[...]


def _edge_mlp(gathered, edges, W1a, W1b, b1, W2, b2):
    BLK = 640
    grid = (N_EDGES // BLK,)
    return pl.pallas_call(
        _mlp_body,
        grid=grid,
        in_specs=[
            pl.BlockSpec((BLK, D_FEAT), lambda i: (i, 0)),
            pl.BlockSpec((BLK, D_EDGE), lambda i: (i, 0)),
            pl.BlockSpec((D_FEAT, D_HID), lambda i: (0, 0)),
            pl.BlockSpec((D_EDGE, D_HID), lambda i: (0, 0)),
            pl.BlockSpec((1, D_HID), lambda i: (0, 0)),
            pl.BlockSpec((D_HID, D_OUT), lambda i: (0, 0)),
            pl.BlockSpec((1, D_OUT), lambda i: (0, 0)),
        ],
        out_specs=pl.BlockSpec((BLK, D_OUT), lambda i: (i, 0)),
        out_shape=jax.ShapeDtypeStruct((N_EDGES, D_OUT), jnp.float32),
    )(gathered, edges, W1a, W1b, b1, W2, b2)


def _smax_body(rec_ref, rows_ref, o_ref):
    EB = rows_ref.shape[0]

    @pl.when(pl.program_id(0) == 0)
    def _():
        o_ref[...] = jnp.full_like(o_ref[...], FMIN)

    def body(j, carry):
        r = rec_ref[0, 0, j]
        row = rows_ref[pl.ds(j, 1), :]
        o_ref[pl.ds(r, 1), :] = jnp.maximum(o_ref[pl.ds(r, 1), :], row)
        return carry

    lax.fori_loop(0, EB, body, 0)


def _segment_max(new_edges, receivers):
    EB = 256
    grid = (N_EDGES // EB,)
    rec3 = receivers.reshape(N_EDGES // EB, 1, EB)
    return pl.pallas_call(
        _smax_body,
        grid=grid,
        in_specs=[
            pl.BlockSpec((1, 1, EB), lambda i: (i, 0, 0), memory_space=pltpu.SMEM),
            pl.BlockSpec((EB, D_OUT), lambda i: (i, 0)),
        ],
        out_specs=pl.BlockSpec((N_NODES, D_OUT), lambda i: (0, 0)),
        out_shape=jax.ShapeDtypeStruct((N_NODES, D_OUT), jnp.float32),
    )(rec3, new_edges)


def kernel(nodes, edges, senders, receivers, W1, b1, W2, b2):
    idx3 = senders.reshape(NW, NCHUNK, CHUNK)
    gathered = _sc_gather(nodes, idx3)
    new_edges = _edge_mlp(
        gathered, edges,
        W1[:D_FEAT], W1[D_FEAT:],
        b1.reshape(1, D_HID), W2, b2.reshape(1, D_OUT),
    )
    return _segment_max(new_edges, receivers)


# speedup vs baseline: 1.3844x; 1.3269x over previous
import functools

import jax
import jax.numpy as jnp
from jax import lax
from jax.experimental import pallas as pl
from jax.experimental.pallas import tpu as pltpu
from jax.experimental.pallas import tpu_sc as plsc

N_NODES = 10000
N_EDGES = 160000
D_FEAT = 256
D_EDGE = 16
D_HID = 512
D_OUT = 256

NC = 2
NS = 16
NW = NC * NS
B_PER_W = N_EDGES // NW
CHUNK = 40
NCHUNK = B_PER_W // CHUNK

FMIN = float(jnp.finfo(jnp.float32).min)


def _sc_gather(table, idx3, n_edges):
    D = table.shape[1]
    b_per_w = n_edges // NW
    nchunk = b_per_w // CHUNK
    mesh = plsc.VectorSubcoreMesh(core_axis_name="c", subcore_axis_name="s")

    NBUF = 4
    NGRP = (nchunk + NBUF - 1) // NBUF

    @functools.partial(
        pl.kernel,
        mesh=mesh,
        out_type=jax.ShapeDtypeStruct((n_edges, D), jnp.float32),
        scratch_types=[
            pltpu.VMEM((nchunk, CHUNK), jnp.int32),
            pltpu.VMEM((CHUNK, D), jnp.float32),
            pltpu.VMEM((CHUNK, D), jnp.float32),
            pltpu.VMEM((CHUNK, D), jnp.float32),
            pltpu.VMEM((CHUNK, D), jnp.float32),
            pltpu.SemaphoreType.DMA,
            pltpu.SemaphoreType.DMA,
            pltpu.SemaphoreType.DMA,
            pltpu.SemaphoreType.DMA,
        ],
    )
    def gather_k(table_hbm, idx_hbm, out_hbm, idx_v,
                 buf0, buf1, buf2, buf3, sem0, sem1, sem2, sem3):
        bufs = (buf0, buf1, buf2, buf3)
        sems = (sem0, sem1, sem2, sem3)
        wid = lax.axis_index("s") * NC + lax.axis_index("c")
        base = wid * b_per_w
        pltpu.sync_copy(idx_hbm.at[wid], idx_v)

        for b in range(NBUF):
            pltpu.async_copy(table_hbm.at[idx_v.at[b]], bufs[b], sems[b])

        def body(g, carry):
            for b in range(NBUF):
                c = g * NBUF + b

                @pl.when(c < nchunk)
                def _():
                    pltpu.make_async_copy(
                        out_hbm.at[pl.ds(0, CHUNK)], bufs[b], sems[b]).wait()
                    pltpu.sync_copy(
                        bufs[b], out_hbm.at[pl.ds(base + c * CHUNK, CHUNK)])

                @pl.when(c + NBUF < nchunk)
                def _():
                    pltpu.async_copy(
                        table_hbm.at[idx_v.at[c + NBUF]], bufs[b], sems[b])
            return carry

        lax.fori_loop(0, NGRP, body, 0)

    return gather_k(table, idx3)


def _mlp_body(g_ref, e_ref, w1a_ref, w1b_ref, b1_ref, w2_ref, b2_ref, o_ref):
    g16 = g_ref[...].astype(jnp.bfloat16)
    w1a16 = w1a_ref[...].astype(jnp.bfloat16)
    h = jnp.dot(g16, w1a16, preferred_element_type=jnp.float32)
    h = h + jnp.dot(e_ref[...], w1b_ref[...], preferred_element_type=jnp.float32)
    h = jnp.maximum(h + b1_ref[...], 0.0)
    o_ref[...] = jnp.dot(h.astype(jnp.bfloat16), w2_ref[...].astype(jnp.bfloat16),
                         preferred_element_type=jnp.float32) + b2_ref[...]


def _edge_mlp(gathered, edges, W1a, W1b, b1, W2, b2, n_edges):
    BLK = 640
    grid = (n_edges // BLK,)
    return pl.pallas_call(
        _mlp_body,
        grid=grid,
        in_specs=[
            pl.BlockSpec((BLK, D_FEAT), lambda i: (i, 0)),
            pl.BlockSpec((BLK, D_EDGE), lambda i: (i, 0)),
            pl.BlockSpec((D_FEAT, D_HID), lambda i: (0, 0)),
            pl.BlockSpec((D_EDGE, D_HID), lambda i: (0, 0)),
            pl.BlockSpec((1, D_HID), lambda i: (0, 0)),
            pl.BlockSpec((D_HID, D_OUT), lambda i: (0, 0)),
            pl.BlockSpec((1, D_OUT), lambda i: (0, 0)),
        ],
        out_specs=pl.BlockSpec((BLK, D_OUT), lambda i: (i, 0)),
        out_shape=jax.ShapeDtypeStruct((n_edges, D_OUT), jnp.float32),
    )(gathered, edges, W1a, W1b, b1, W2, b2)


def _smax_body(rec_ref, rows_ref, o_ref):
    EB = rows_ref.shape[0]
    U = 8

    @pl.when(pl.program_id(0) == 0)
    def _():
        o_ref[...] = jnp.full_like(o_ref[...], FMIN)

    def body(jj, carry):
        j = jj * U
        rows = [rows_ref[pl.ds(j + u, 1), :] for u in range(U)]
        for u in range(U):
            r = rec_ref[0, 0, j + u]
            o_ref[pl.ds(r, 1), :] = jnp.maximum(o_ref[pl.ds(r, 1), :], rows[u])
        return carry

    lax.fori_loop(0, EB // U, body, 0)


def _segment_max_tc(new_edges, receivers):
    EB = 256
    grid = (N_EDGES // EB,)
    rec3 = receivers.reshape(N_EDGES // EB, 1, EB)
    return pl.pallas_call(
        _smax_body,
        grid=grid,
        in_specs=[
            pl.BlockSpec((1, 1, EB), lambda i: (i, 0, 0), memory_space=pltpu.SMEM),
            pl.BlockSpec((EB, D_OUT), lambda i: (i, 0)),
        ],
        out_specs=pl.BlockSpec((N_NODES, D_OUT), lambda i: (0, 0)),
        out_shape=jax.ShapeDtypeStruct((N_NODES, D_OUT), jnp.float32),
    )(rec3, new_edges)


SPLIT = 5
EC = N_EDGES // SPLIT


def kernel(nodes, edges, senders, receivers, W1, b1, W2, b2):
    W1a, W1b = W1[:D_FEAT], W1[D_FEAT:]
    b1r, b2r = b1.reshape(1, D_HID), b2.reshape(1, D_OUT)
    outs = []
    for s in range(SPLIT):
        sl = slice(s * EC, (s + 1) * EC)
        idx3 = senders[sl].reshape(NW, EC // NW // CHUNK, CHUNK)
        g = _sc_gather(nodes, idx3, EC)
        outs.append(_edge_mlp(g, edges[sl], W1a, W1b, b1r, W2, b2r, EC))
    new_edges = jnp.concatenate(outs, axis=0)
    return _segment_max_tc(new_edges, receivers)


# speedup vs baseline: 1.5247x; 1.1014x over previous
import functools

import jax
import jax.numpy as jnp
from jax import lax
from jax.experimental import pallas as pl
from jax.experimental.pallas import tpu as pltpu
from jax.experimental.pallas import tpu_sc as plsc

N_NODES = 10000
N_EDGES = 160000
D_FEAT = 256
D_EDGE = 16
D_HID = 512
D_OUT = 256

NC = 2
NS = 16
NW = NC * NS
B_PER_W = N_EDGES // NW
CHUNK = 40
NCHUNK = B_PER_W // CHUNK

FMIN = float(jnp.finfo(jnp.float32).min)


def _sc_gather(table, idx3, n_edges):
    D = table.shape[1]
    b_per_w = n_edges // NW
    nchunk = b_per_w // CHUNK
    mesh = plsc.VectorSubcoreMesh(core_axis_name="c", subcore_axis_name="s")

    NBUF = 4
    NGRP = (nchunk + NBUF - 1) // NBUF

    @functools.partial(
        pl.kernel,
        mesh=mesh,
        out_type=jax.ShapeDtypeStruct((n_edges, D), jnp.float32),
        scratch_types=[
            pltpu.VMEM((nchunk, CHUNK), jnp.int32),
            pltpu.VMEM((CHUNK, D), jnp.float32),
            pltpu.VMEM((CHUNK, D), jnp.float32),
            pltpu.VMEM((CHUNK, D), jnp.float32),
            pltpu.VMEM((CHUNK, D), jnp.float32),
            pltpu.SemaphoreType.DMA,
            pltpu.SemaphoreType.DMA,
            pltpu.SemaphoreType.DMA,
            pltpu.SemaphoreType.DMA,
        ],
    )
    def gather_k(table_hbm, idx_hbm, out_hbm, idx_v,
                 buf0, buf1, buf2, buf3, sem0, sem1, sem2, sem3):
        bufs = (buf0, buf1, buf2, buf3)
        sems = (sem0, sem1, sem2, sem3)
        wid = lax.axis_index("s") * NC + lax.axis_index("c")
        base = wid * b_per_w
        pltpu.sync_copy(idx_hbm.at[wid], idx_v)

        for b in range(NBUF):
            pltpu.async_copy(table_hbm.at[idx_v.at[b]], bufs[b], sems[b])

        def body(g, carry):
            for b in range(NBUF):
                c = g * NBUF + b

                @pl.when(c < nchunk)
                def _():
                    pltpu.make_async_copy(
                        out_hbm.at[pl.ds(0, CHUNK)], bufs[b], sems[b]).wait()
                    pltpu.sync_copy(
                        bufs[b], out_hbm.at[pl.ds(base + c * CHUNK, CHUNK)])

                @pl.when(c + NBUF < nchunk)
                def _():
                    pltpu.async_copy(
                        table_hbm.at[idx_v.at[c + NBUF]], bufs[b], sems[b])
            return carry

        lax.fori_loop(0, NGRP, body, 0)

    return gather_k(table, idx3)


def _mlp_body(g_ref, e_ref, w1a_ref, w1b_ref, b1_ref, w2_ref, b2_ref, o_ref):
    g16 = g_ref[...].astype(jnp.bfloat16)
    w1a16 = w1a_ref[...].astype(jnp.bfloat16)
    h = jnp.dot(g16, w1a16, preferred_element_type=jnp.float32)
    h = h + jnp.dot(e_ref[...], w1b_ref[...], preferred_element_type=jnp.float32)
    h = jnp.maximum(h + b1_ref[...], 0.0)
    o_ref[...] = jnp.dot(h.astype(jnp.bfloat16), w2_ref[...].astype(jnp.bfloat16),
                         preferred_element_type=jnp.float32) + b2_ref[...]


def _edge_mlp(gathered, edges, W1a, W1b, b1, W2, b2, n_edges):
    BLK = 640
    grid = (n_edges // BLK,)
    return pl.pallas_call(
        _mlp_body,
        grid=grid,
        in_specs=[
            pl.BlockSpec((BLK, D_FEAT), lambda i: (i, 0)),
            pl.BlockSpec((BLK, D_EDGE), lambda i: (i, 0)),
            pl.BlockSpec((D_FEAT, D_HID), lambda i: (0, 0)),
            pl.BlockSpec((D_EDGE, D_HID), lambda i: (0, 0)),
            pl.BlockSpec((1, D_HID), lambda i: (0, 0)),
            pl.BlockSpec((D_HID, D_OUT), lambda i: (0, 0)),
            pl.BlockSpec((1, D_OUT), lambda i: (0, 0)),
        ],
        out_specs=pl.BlockSpec((BLK, D_OUT), lambda i: (i, 0)),
        out_shape=jax.ShapeDtypeStruct((n_edges, D_OUT), jnp.float32),
    )(gathered, edges, W1a, W1b, b1, W2, b2)


def _smax_body(rec_ref, rows_ref, o_ref, a1, a2, a3):
    EB = rows_ref.shape[0]
    U = 8
    accs = (o_ref, a1, a2, a3)

    @pl.when(pl.program_id(0) == 0)
    def _():
        for a in accs:
            a[...] = jnp.full_like(o_ref[...], FMIN)

    def body(jj, carry):
        j = jj * U
        rows = [rows_ref[pl.ds(j + u, 1), :] for u in range(U)]
        for u in range(U):
            r = rec_ref[0, 0, j + u]
            a = accs[u % 4]
            a[pl.ds(r, 1), :] = jnp.maximum(a[pl.ds(r, 1), :], rows[u])
        return carry

    lax.fori_loop(0, EB // U, body, 0)

    @pl.when(pl.program_id(0) == pl.num_programs(0) - 1)
    def _():
        o_ref[...] = jnp.maximum(jnp.maximum(o_ref[...], a1[...]),
                                 jnp.maximum(a2[...], a3[...]))


def _segment_max_tc(new_edges, receivers):
    EB = 256
    grid = (N_EDGES // EB,)
    rec3 = receivers.reshape(N_EDGES // EB, 1, EB)
    return pl.pallas_call(
        _smax_body,
        grid=grid,
        in_specs=[
            pl.BlockSpec((1, 1, EB), lambda i: (i, 0, 0), memory_space=pltpu.SMEM),
            pl.BlockSpec((EB, D_OUT), lambda i: (i, 0)),
        ],
        out_specs=pl.BlockSpec((N_NODES, D_OUT), lambda i: (0, 0)),
        out_shape=jax.ShapeDtypeStruct((N_NODES, D_OUT), jnp.float32),
        scratch_shapes=[
            pltpu.VMEM((N_NODES, D_OUT), jnp.float32),
            pltpu.VMEM((N_NODES, D_OUT), jnp.float32),
            pltpu.VMEM((N_NODES, D_OUT), jnp.float32),
        ],
    )(rec3, new_edges)


SPLIT = 5
EC = N_EDGES // SPLIT


def kernel(nodes, edges, senders, receivers, W1, b1, W2, b2):
    W1a, W1b = W1[:D_FEAT], W1[D_FEAT:]
    b1r, b2r = b1.reshape(1, D_HID), b2.reshape(1, D_OUT)
    outs = []
    for s in range(SPLIT):
        sl = slice(s * EC, (s + 1) * EC)
        idx3 = senders[sl].reshape(NW, EC // NW // CHUNK, CHUNK)
        g = _sc_gather(nodes, idx3, EC)
        outs.append(_edge_mlp(g, edges[sl], W1a, W1b, b1r, W2, b2r, EC))
    new_edges = jnp.concatenate(outs, axis=0)
    return _segment_max_tc(new_edges, receivers)


# speedup vs baseline: 1.5804x; 1.0365x over previous
import functools

import jax
import jax.numpy as jnp
from jax import lax
from jax.experimental import pallas as pl
from jax.experimental.pallas import tpu as pltpu
from jax.experimental.pallas import tpu_sc as plsc

N_NODES = 10000
N_EDGES = 160000
D_FEAT = 256
D_EDGE = 16
D_HID = 512
D_OUT = 256

NC = 2
NS = 16
NW = NC * NS
B_PER_W = N_EDGES // NW
CHUNK = 40
NCHUNK = B_PER_W // CHUNK

FMIN = float(jnp.finfo(jnp.float32).min)


def _sc_gather(table, idx3, n_edges):
    D = table.shape[1]
    b_per_w = n_edges // NW
    nchunk = b_per_w // CHUNK
    mesh = plsc.VectorSubcoreMesh(core_axis_name="c", subcore_axis_name="s")

    NBUF = 4
    NGRP = (nchunk + NBUF - 1) // NBUF

    @functools.partial(
        pl.kernel,
        mesh=mesh,
        out_type=jax.ShapeDtypeStruct((n_edges, D), jnp.float32),
        scratch_types=[
            pltpu.VMEM((nchunk, CHUNK), jnp.int32),
            pltpu.VMEM((CHUNK, D), jnp.float32),
            pltpu.VMEM((CHUNK, D), jnp.float32),
            pltpu.VMEM((CHUNK, D), jnp.float32),
            pltpu.VMEM((CHUNK, D), jnp.float32),
            pltpu.SemaphoreType.DMA,
            pltpu.SemaphoreType.DMA,
            pltpu.SemaphoreType.DMA,
            pltpu.SemaphoreType.DMA,
        ],
    )
    def gather_k(table_hbm, idx_hbm, out_hbm, idx_v,
                 buf0, buf1, buf2, buf3, sem0, sem1, sem2, sem3):
        bufs = (buf0, buf1, buf2, buf3)
        sems = (sem0, sem1, sem2, sem3)
        wid = lax.axis_index("s") * NC + lax.axis_index("c")
        base = wid * b_per_w
        pltpu.sync_copy(idx_hbm.at[wid], idx_v)

        for b in range(NBUF):
            pltpu.async_copy(table_hbm.at[idx_v.at[b]], bufs[b], sems[b])

        def body(g, carry):
            for b in range(NBUF):
                c = g * NBUF + b

                @pl.when(c < nchunk)
                def _():
                    pltpu.make_async_copy(
                        out_hbm.at[pl.ds(0, CHUNK)], bufs[b], sems[b]).wait()
                    pltpu.sync_copy(
                        bufs[b], out_hbm.at[pl.ds(base + c * CHUNK, CHUNK)])

                @pl.when(c + NBUF < nchunk)
                def _():
                    pltpu.async_copy(
                        table_hbm.at[idx_v.at[c + NBUF]], bufs[b], sems[b])
            return carry

        lax.fori_loop(0, NGRP, body, 0)

    return gather_k(table, idx3)


def _mlp_body(g_ref, e_ref, w1a_ref, w1b_ref, b1_ref, w2_ref, b2_ref, o_ref):
    g16 = g_ref[...].astype(jnp.bfloat16)
    w1a16 = w1a_ref[...].astype(jnp.bfloat16)
    h = jnp.dot(g16, w1a16, preferred_element_type=jnp.float32)
    h = h + jnp.dot(e_ref[...], w1b_ref[...], preferred_element_type=jnp.float32)
    h = jnp.maximum(h + b1_ref[...], 0.0)
    o_ref[...] = jnp.dot(h.astype(jnp.bfloat16), w2_ref[...].astype(jnp.bfloat16),
                         preferred_element_type=jnp.float32) + b2_ref[...]


def _edge_mlp(gathered, edges, W1a, W1b, b1, W2, b2, n_edges):
    BLK = 640
    grid = (n_edges // BLK,)
    return pl.pallas_call(
        _mlp_body,
        grid=grid,
        in_specs=[
            pl.BlockSpec((BLK, D_FEAT), lambda i: (i, 0)),
            pl.BlockSpec((BLK, D_EDGE), lambda i: (i, 0)),
            pl.BlockSpec((D_FEAT, D_HID), lambda i: (0, 0)),
            pl.BlockSpec((D_EDGE, D_HID), lambda i: (0, 0)),
            pl.BlockSpec((1, D_HID), lambda i: (0, 0)),
            pl.BlockSpec((D_HID, D_OUT), lambda i: (0, 0)),
            pl.BlockSpec((1, D_OUT), lambda i: (0, 0)),
        ],
        out_specs=pl.BlockSpec((BLK, D_OUT), lambda i: (i, 0)),
        out_shape=jax.ShapeDtypeStruct((n_edges, D_OUT), jnp.float32),
    )(gathered, edges, W1a, W1b, b1, W2, b2)


def _smax_body(rec_ref, rows_ref, o_ref, a1, a2, a3):
    EB = rows_ref.shape[0]
    U = 16
    accs = (o_ref, a1, a2, a3)

    @pl.when(pl.program_id(0) == 0)
    def _():
        for a in accs:
            a[...] = jnp.full_like(o_ref[...], FMIN)

    def body(jj, carry):
        j = jj * U
        rows = [rows_ref[pl.ds(j + u, 1), :] for u in range(U)]
        for u in range(U):
            r = rec_ref[0, 0, j + u]
            a = accs[u % 4]
            a[pl.ds(r, 1), :] = jnp.maximum(a[pl.ds(r, 1), :], rows[u])
        return carry

    lax.fori_loop(0, EB // U, body, 0)

    @pl.when(pl.program_id(0) == pl.num_programs(0) - 1)
    def _():
        o_ref[...] = jnp.maximum(jnp.maximum(o_ref[...], a1[...]),
                                 jnp.maximum(a2[...], a3[...]))


def _segment_max_tc(new_edges, receivers):
    EB = 320
    grid = (N_EDGES // EB,)
    rec3 = receivers.reshape(N_EDGES // EB, 1, EB)
    return pl.pallas_call(
        _smax_body,
        grid=grid,
        in_specs=[
            pl.BlockSpec((1, 1, EB), lambda i: (i, 0, 0), memory_space=pltpu.SMEM),
            pl.BlockSpec((EB, D_OUT), lambda i: (i, 0)),
        ],
        out_specs=pl.BlockSpec((N_NODES, D_OUT), lambda i: (0, 0)),
        out_shape=jax.ShapeDtypeStruct((N_NODES, D_OUT), jnp.float32),
        scratch_shapes=[
            pltpu.VMEM((N_NODES, D_OUT), jnp.float32),
            pltpu.VMEM((N_NODES, D_OUT), jnp.float32),
            pltpu.VMEM((N_NODES, D_OUT), jnp.float32),
        ],
    )(rec3, new_edges)


SPLIT = 5
EC = N_EDGES // SPLIT


def kernel(nodes, edges, senders, receivers, W1, b1, W2, b2):
    W1a, W1b = W1[:D_FEAT], W1[D_FEAT:]
    b1r, b2r = b1.reshape(1, D_HID), b2.reshape(1, D_OUT)
    outs = []
    for s in range(SPLIT):
        sl = slice(s * EC, (s + 1) * EC)
        idx3 = senders[sl].reshape(NW, EC // NW // CHUNK, CHUNK)
        g = _sc_gather(nodes, idx3, EC)
        outs.append(_edge_mlp(g, edges[sl], W1a, W1b, b1r, W2, b2r, EC))
    new_edges = jnp.concatenate(outs, axis=0)
    return _segment_max_tc(new_edges, receivers)
